# baseline (speedup 1.0000x reference)
import jax
import jax.numpy as jnp
from jax import lax
from jax.experimental import pallas as pl
from jax.experimental.pallas import tpu as pltpu
from jax.experimental.pallas import tpu_sc as plsc

_RADIUS = 0.2
_NSAMPLE = 16
_N = 8192
_NQ = 2048
_B = 2
_CR = 128
_L = 16
_NC, _NSUB = 2, 16
_NW = _NC * _NSUB
_QPW = (_B * _NQ) // _NW
_NVREG = _N // _L


def _body(p_h, q_h, f_h,
          nf_h, gx_h, bq_h,
          pflat, pxv, pyv, pzv, qflat, idxg, cand, bqs,
          planeA, planeB, stageA, stageB,
          semLA, semLB, semSA, semSB, semW,
          gxx, gxy, gxz, xfx, xfy, xfz):
    cid = lax.axis_index("c")
    sid = lax.axis_index("s")
    wid = sid * _NC + cid
    q0 = wid * _QPW
    b = q0 // _NQ
    qb = q0 - b * _NQ

    pltpu.sync_copy(p_h.at[b], pflat)
    pltpu.sync_copy(q_h.at[b, pl.ds(qb, _QPW)], qflat)

    lane = jnp.arange(_L, dtype=jnp.int32)
    r2 = jnp.float32(_RADIUS * _RADIUS)

    def _full(i):
        return jnp.full((_L,), i, dtype=jnp.int32)

    def _splat(ref2d, i, c):
        return plsc.load_gather(ref2d, [_full(i), _full(c)])

    def deint(v, carry):
        base = v * _L
        rows = lane + base
        pxv[pl.ds(base, _L)] = plsc.load_gather(pflat, [rows, _full(0)])
        pyv[pl.ds(base, _L)] = plsc.load_gather(pflat, [rows, _full(1)])
        pzv[pl.ds(base, _L)] = plsc.load_gather(pflat, [rows, _full(2)])
        return carry

    lax.fori_loop(0, _NVREG, deint, jnp.int32(0), unroll=4)

    shadow = jnp.full((_L,), 1000000.0, dtype=jnp.float32)
    pxv[pl.ds(_N, _L)] = shadow
    pyv[pl.ds(_N, _L)] = shadow
    pzv[pl.ds(_N, _L)] = shadow
    zeros = jnp.zeros((_L,), dtype=jnp.float32)
    planeA[pl.ds(_N, _L)] = zeros
    planeB[pl.ds(_N, _L)] = zeros

    _U = 4
    _NG = _NVREG // _U

    def ballq(i, carry):
        qx = _splat(qflat, i, 0)
        qy = _splat(qflat, i, 1)
        qz = _splat(qflat, i, 2)

        def w_cond(st):
            g, cnt = st
            return jnp.logical_and(g < _NG, cnt < _NSAMPLE)

        def w_body(st):
            g, cnt = st
            base0 = g * (_U * _L)
            for u in range(_U):
                base = base0 + u * _L
                px = pxv[pl.ds(base, _L)]
                py = pyv[pl.ds(base, _L)]
                pz = pzv[pl.ds(base, _L)]
                dx = qx - px
                dy = qy - py
                dz = qz - pz
                d2 = dx * dx + dy * dy + dz * dz
                m = d2 < r2
                plsc.store_compressed(cand.at[pl.ds(cnt, _L)],
                                      lane + base, mask=m)
                pc = plsc.all_reduce_population_count(m)
                cnt = cnt + pc[0]
            return g + 1, cnt

        _, cnt = lax.while_loop(
            w_cond, w_body, (jnp.int32(0), jnp.int32(0)))

        row = cand[pl.ds(0, _L)]
        firstv = plsc.load_gather(cand, [jnp.zeros((_L,), jnp.int32)])
        first = jnp.where(cnt > 0, firstv, jnp.int32(0))
        bq_row = jnp.where(lane < cnt, row, first)
        pad = jnp.logical_and(lane > 0, lane >= cnt)
        bqs[i] = bq_row
        idxg[pl.ds(i * _L, _L)] = jnp.where(pad, jnp.int32(_N), bq_row)
        return carry

    lax.fori_loop(0, _QPW, ballq, jnp.int32(0))
    pltpu.async_copy(bqs, bq_h.at[b, pl.ds(qb, _QPW)], semW)

    def xyzg(i, carry):
        idxv = idxg[pl.ds(i * _L, _L)]
        qx = _splat(qflat, i, 0)
        qy = _splat(qflat, i, 1)
        qz = _splat(qflat, i, 2)
        gx = plsc.load_gather(pxv, [idxv]) - qx
        gy = plsc.load_gather(pyv, [idxv]) - qy
        gz = plsc.load_gather(pzv, [idxv]) - qz
        gxx[i] = gx
        gxy[i] = gy
        gxz[i] = gz
        thr = jnp.float32(100000.0)
        zero = jnp.float32(0.0)
        rad = jnp.float32(_RADIUS)
        xfx[i] = jnp.where(gx > thr, zero, gx) / rad
        xfy[i] = jnp.where(gy > thr, zero, gy) / rad
        xfz[i] = jnp.where(gz > thr, zero, gz) / rad
        return carry

    lax.fori_loop(0, _QPW, xyzg, jnp.int32(0), unroll=2)
    qsl = pl.ds(qb, _QPW)
    pltpu.async_copy(gxx, gx_h.at[b, 0, qsl], semW)
    pltpu.async_copy(gxy, gx_h.at[b, 1, qsl], semW)
    pltpu.async_copy(gxz, gx_h.at[b, 2, qsl], semW)
    for r in range(4):
        pltpu.async_copy(xfx, nf_h.at[b, 0, r, qsl], semW)
        pltpu.async_copy(xfy, nf_h.at[b, 1, r, qsl], semW)
        pltpu.async_copy(xfz, nf_h.at[b, 2, r, qsl], semW)

    def gq(plane, stage):
        def qloop(i, c2):
            idxv = idxg[pl.ds(i * _L, _L)]
            stage[i] = plsc.load_gather(plane, [idxv])
            return c2
        lax.fori_loop(0, _QPW, qloop, jnp.int32(0), unroll=4)

    def _nf_dst(p):
        return nf_h.at[b, 3 + p // 4, p % 4, pl.ds(qb, _QPW)]

    def _pl_src(p):
        return f_h.at[b, p // 4, p % 4]

    npair = _CR // 2
    pltpu.async_copy(_pl_src(0), planeA.at[pl.ds(0, _N)], semLA)

    def pair(j, carry):
        p = j * 2
        pltpu.make_async_copy(_pl_src(p), planeA.at[pl.ds(0, _N)],
                              semLA).wait()
        pltpu.async_copy(_pl_src(p + 1), planeB.at[pl.ds(0, _N)], semLB)

        @pl.when(j > 0)
        def _():
            pltpu.make_async_copy(stageA, _nf_dst(p - 2), semSA).wait()

        gq(planeA, stageA)
        pltpu.async_copy(stageA, _nf_dst(p), semSA)

        pltpu.make_async_copy(_pl_src(p + 1), planeB.at[pl.ds(0, _N)],
                              semLB).wait()

        @pl.when(j + 1 < npair)
        def _():
            pltpu.async_copy(_pl_src(p + 2), planeA.at[pl.ds(0, _N)], semLA)

        @pl.when(j > 0)
        def _():
            pltpu.make_async_copy(stageB, _nf_dst(p - 1), semSB).wait()

        gq(planeB, stageB)
        pltpu.async_copy(stageB, _nf_dst(p + 1), semSB)
        return carry

    lax.fori_loop(0, npair, pair, jnp.int32(0))

    pltpu.make_async_copy(stageA, _nf_dst(_CR - 2), semSA).wait()
    pltpu.make_async_copy(stageB, _nf_dst(_CR - 1), semSB).wait()
    pltpu.make_async_copy(bqs, bq_h.at[b, pl.ds(qb, _QPW)], semW).wait()
    pltpu.make_async_copy(gxx, gx_h.at[b, 0, qsl], semW).wait()
    pltpu.make_async_copy(gxy, gx_h.at[b, 1, qsl], semW).wait()
    pltpu.make_async_copy(gxz, gx_h.at[b, 2, qsl], semW).wait()
    for r in range(4):
        pltpu.make_async_copy(xfx, nf_h.at[b, 0, r, qsl], semW).wait()
        pltpu.make_async_copy(xfy, nf_h.at[b, 1, r, qsl], semW).wait()
        pltpu.make_async_copy(xfz, nf_h.at[b, 2, r, qsl], semW).wait()


def kernel(xyz, new_xyz, features):
    B, N, _ = xyz.shape

    mesh = plsc.VectorSubcoreMesh(
        core_axis_name="c", subcore_axis_name="s",
        num_cores=_NC, num_subcores=_NSUB)

    f32 = jnp.float32
    out_type = (
        jax.ShapeDtypeStruct((B, 35, 4, _NQ, _NSAMPLE), f32),
        jax.ShapeDtypeStruct((B, 3, _NQ, _NSAMPLE), f32),
        jax.ShapeDtypeStruct((B, _NQ, _NSAMPLE), jnp.int32),
    )
    scratch_types = [
        pltpu.VMEM((N, 3), f32),
        pltpu.VMEM((N + _L,), f32),
        pltpu.VMEM((N + _L,), f32),
        pltpu.VMEM((N + _L,), f32),
        pltpu.VMEM((_QPW, 3), f32),
        pltpu.VMEM((_QPW * _L,), jnp.int32),
        pltpu.VMEM((96,), jnp.int32),
        pltpu.VMEM((_QPW, _L), jnp.int32),
        pltpu.VMEM((N + _L,), f32),
        pltpu.VMEM((N + _L,), f32),
        pltpu.VMEM((_QPW, _L), f32),
        pltpu.VMEM((_QPW, _L), f32),
        pltpu.SemaphoreType.DMA,
        pltpu.SemaphoreType.DMA,
        pltpu.SemaphoreType.DMA,
        pltpu.SemaphoreType.DMA,
        pltpu.SemaphoreType.DMA,
        pltpu.VMEM((_QPW, _L), f32),
        pltpu.VMEM((_QPW, _L), f32),
        pltpu.VMEM((_QPW, _L), f32),
        pltpu.VMEM((_QPW, _L), f32),
        pltpu.VMEM((_QPW, _L), f32),
        pltpu.VMEM((_QPW, _L), f32),
    ]
    run = pl.kernel(_body, out_type=out_type, mesh=mesh,
                    scratch_types=scratch_types,
                    compiler_params=pltpu.CompilerParams(
                        needs_layout_passes=False,
                        use_tc_tiling_on_sc=False))
    return run(xyz, new_xyz, features)


# speedup vs baseline: 1.0254x; 1.0254x over previous
import jax
import jax.numpy as jnp
from jax import lax
from jax.experimental import pallas as pl
from jax.experimental.pallas import tpu as pltpu
from jax.experimental.pallas import tpu_sc as plsc

_RADIUS = 0.2
_NSAMPLE = 16
_N = 8192
_NQ = 2048
_B = 2
_CR = 128
_L = 16
_NC, _NSUB = 2, 16
_NW = _NC * _NSUB
_QPW = (_B * _NQ) // _NW
_NVREG = _N // _L
_HALF = _CR // 2


def _body(p_h, q_h, f_h,
          nf_h, gx_h, bq_h, ig_h,
          pflat, pxv, pyv, pzv, qflat, idxg, idxg2, cand, bqs,
          planeA, planeB, stA, stA2, stB, stB2,
          semLA, semLB, semSA, semSB, semW,
          gxx, gxy, gxz, xfx, xfy, xfz):
    cid = lax.axis_index("c")
    sid = lax.axis_index("s")
    wid = sid * _NC + cid
    q0 = wid * _QPW
    b = q0 // _NQ
    qb = q0 - b * _NQ
    widp = (sid ^ 1) * _NC + cid
    qbp = widp * _QPW - b * _NQ

    pltpu.sync_copy(p_h.at[b], pflat)
    pltpu.sync_copy(q_h.at[b, pl.ds(qb * 3, _QPW * 3)], qflat)

    lane = jnp.arange(_L, dtype=jnp.int32)
    r2 = jnp.float32(_RADIUS * _RADIUS)

    def _splat(ref, i):
        return plsc.load_gather(ref, [jnp.full((_L,), i, dtype=jnp.int32)])

    def deint(v, carry):
        base = v * _L
        i3 = (lane + base) * 3
        pxv[pl.ds(base, _L)] = plsc.load_gather(pflat, [i3])
        pyv[pl.ds(base, _L)] = plsc.load_gather(pflat, [i3 + 1])
        pzv[pl.ds(base, _L)] = plsc.load_gather(pflat, [i3 + 2])
        return carry

    lax.fori_loop(0, _NVREG, deint, jnp.int32(0), unroll=4)

    shadow = jnp.full((_L,), 1000000.0, dtype=jnp.float32)
    pxv[pl.ds(_N, _L)] = shadow
    pyv[pl.ds(_N, _L)] = shadow
    pzv[pl.ds(_N, _L)] = shadow
    zeros = jnp.zeros((_L,), dtype=jnp.float32)
    planeA[pl.ds(_N, _L)] = zeros
    planeB[pl.ds(_N, _L)] = zeros

    _U = 4
    _NG = _NVREG // _U

    def ballq(i, carry):
        qx = _splat(qflat, 3 * i)
        qy = _splat(qflat, 3 * i + 1)
        qz = _splat(qflat, 3 * i + 2)

        def w_cond(st):
            g, cnt = st
            return jnp.logical_and(g < _NG, cnt < _NSAMPLE)

        def w_body(st):
            g, cnt = st
            base0 = g * (_U * _L)
            for u in range(_U):
                base = base0 + u * _L
                px = pxv[pl.ds(base, _L)]
                py = pyv[pl.ds(base, _L)]
                pz = pzv[pl.ds(base, _L)]
                dx = qx - px
                dy = qy - py
                dz = qz - pz
                d2 = dx * dx + dy * dy + dz * dz
                m = d2 < r2
                plsc.store_compressed(cand.at[pl.ds(cnt, _L)],
                                      lane + base, mask=m)
                pc = plsc.all_reduce_population_count(m)
                cnt = cnt + pc[0]
            return g + 1, cnt

        _, cnt = lax.while_loop(
            w_cond, w_body, (jnp.int32(0), jnp.int32(0)))

        row = cand[pl.ds(0, _L)]
        firstv = _splat(cand, jnp.int32(0))
        first = jnp.where(cnt > 0, firstv, jnp.int32(0))
        bq_row = jnp.where(lane < cnt, row, first)
        pad = jnp.logical_and(lane > 0, lane >= cnt)
        bqs[i] = bq_row
        idxg[i] = jnp.where(pad, jnp.int32(_N), bq_row)
        return carry

    lax.fori_loop(0, _QPW, ballq, jnp.int32(0))
    pltpu.async_copy(bqs, bq_h.at[b, pl.ds(qb, _QPW)], semW)
    pltpu.sync_copy(idxg, ig_h.at[b, pl.ds(qb, _QPW)])

    def xyzg(i, carry):
        idxv = idxg[i]
        qx = _splat(qflat, 3 * i)
        qy = _splat(qflat, 3 * i + 1)
        qz = _splat(qflat, 3 * i + 2)
        gx = plsc.load_gather(pxv, [idxv]) - qx
        gy = plsc.load_gather(pyv, [idxv]) - qy
        gz = plsc.load_gather(pzv, [idxv]) - qz
        gxx[i] = gx
        gxy[i] = gy
        gxz[i] = gz
        thr = jnp.float32(100000.0)
        zero = jnp.float32(0.0)
        rad = jnp.float32(_RADIUS)
        xfx[i] = jnp.where(gx > thr, zero, gx) / rad
        xfy[i] = jnp.where(gy > thr, zero, gy) / rad
        xfz[i] = jnp.where(gz > thr, zero, gz) / rad
        return carry

    lax.fori_loop(0, _QPW, xyzg, jnp.int32(0), unroll=2)
    qsl = pl.ds(qb, _QPW)
    pltpu.async_copy(gxx, gx_h.at[b, 0, qsl], semW)
    pltpu.async_copy(gxy, gx_h.at[b, 1, qsl], semW)
    pltpu.async_copy(gxz, gx_h.at[b, 2, qsl], semW)
    for r in range(4):
        pltpu.async_copy(xfx, nf_h.at[b, 0, r, qsl], semW)
        pltpu.async_copy(xfy, nf_h.at[b, 1, r, qsl], semW)
        pltpu.async_copy(xfz, nf_h.at[b, 2, r, qsl], semW)

    plsc.subcore_barrier()
    pltpu.sync_copy(ig_h.at[b, pl.ds(qbp, _QPW)], idxg2)

    def gq(plane, stage, iref):
        def qloop(i, c2):
            stage[i] = plsc.load_gather(plane, [iref[i]])
            return c2
        lax.fori_loop(0, _QPW, qloop, jnp.int32(0), unroll=4)

    pbase = (sid % 2) * _HALF

    def _nf_dst(p, qoff):
        return nf_h.at[b, 3 + p // 4, p % 4, pl.ds(qoff, _QPW)]

    def _pl_src(p):
        return f_h.at[b, p]

    npair = _HALF // 2
    pltpu.async_copy(_pl_src(pbase), planeA.at[pl.ds(0, _N)], semLA)

    def pair(j, carry):
        p = pbase + j * 2
        pltpu.make_async_copy(_pl_src(p), planeA.at[pl.ds(0, _N)],
                              semLA).wait()
        pltpu.async_copy(_pl_src(p + 1), planeB.at[pl.ds(0, _N)], semLB)

        @pl.when(j > 0)
        def _():
            pltpu.make_async_copy(stA, _nf_dst(p - 2, qb), semSA).wait()
            pltpu.make_async_copy(stA2, _nf_dst(p - 2, qbp), semSA).wait()

        gq(planeA, stA, idxg)
        gq(planeA, stA2, idxg2)
        pltpu.async_copy(stA, _nf_dst(p, qb), semSA)
        pltpu.async_copy(stA2, _nf_dst(p, qbp), semSA)

        pltpu.make_async_copy(_pl_src(p + 1), planeB.at[pl.ds(0, _N)],
                              semLB).wait()

        @pl.when(j + 1 < npair)
        def _():
            pltpu.async_copy(_pl_src(p + 2), planeA.at[pl.ds(0, _N)], semLA)

        @pl.when(j > 0)
        def _():
            pltpu.make_async_copy(stB, _nf_dst(p - 1, qb), semSB).wait()
            pltpu.make_async_copy(stB2, _nf_dst(p - 1, qbp), semSB).wait()

        gq(planeB, stB, idxg)
        gq(planeB, stB2, idxg2)
        pltpu.async_copy(stB, _nf_dst(p + 1, qb), semSB)
        pltpu.async_copy(stB2, _nf_dst(p + 1, qbp), semSB)
        return carry

    lax.fori_loop(0, npair, pair, jnp.int32(0))

    plast = pbase + _HALF - 2
    pltpu.make_async_copy(stA, _nf_dst(plast, qb), semSA).wait()
    pltpu.make_async_copy(stA2, _nf_dst(plast, qbp), semSA).wait()
    pltpu.make_async_copy(stB, _nf_dst(plast + 1, qb), semSB).wait()
    pltpu.make_async_copy(stB2, _nf_dst(plast + 1, qbp), semSB).wait()
    pltpu.make_async_copy(bqs, bq_h.at[b, pl.ds(qb, _QPW)], semW).wait()
    pltpu.make_async_copy(gxx, gx_h.at[b, 0, qsl], semW).wait()
    pltpu.make_async_copy(gxy, gx_h.at[b, 1, qsl], semW).wait()
    pltpu.make_async_copy(gxz, gx_h.at[b, 2, qsl], semW).wait()
    for r in range(4):
        pltpu.make_async_copy(xfx, nf_h.at[b, 0, r, qsl], semW).wait()
        pltpu.make_async_copy(xfy, nf_h.at[b, 1, r, qsl], semW).wait()
        pltpu.make_async_copy(xfz, nf_h.at[b, 2, r, qsl], semW).wait()


def kernel(xyz, new_xyz, features):
    B, N, _ = xyz.shape
    pts = xyz.reshape(B, N * 3)
    qs = new_xyz.reshape(B, _NQ * 3)
    feats = features.reshape(B, _CR, N)

    mesh = plsc.VectorSubcoreMesh(
        core_axis_name="c", subcore_axis_name="s",
        num_cores=_NC, num_subcores=_NSUB)

    f32 = jnp.float32
    out_type = (
        jax.ShapeDtypeStruct((B, 35, 4, _NQ, _NSAMPLE), f32),
        jax.ShapeDtypeStruct((B, 3, _NQ, _NSAMPLE), f32),
        jax.ShapeDtypeStruct((B, _NQ, _NSAMPLE), jnp.int32),
        jax.ShapeDtypeStruct((B, _NQ, _NSAMPLE), jnp.int32),
    )
    scratch_types = [
        pltpu.VMEM((N * 3,), f32),
        pltpu.VMEM((N + _L,), f32),
        pltpu.VMEM((N + _L,), f32),
        pltpu.VMEM((N + _L,), f32),
        pltpu.VMEM((_QPW * 3,), f32),
        pltpu.VMEM((_QPW, _L), jnp.int32),
        pltpu.VMEM((_QPW, _L), jnp.int32),
        pltpu.VMEM((96,), jnp.int32),
        pltpu.VMEM((_QPW, _L), jnp.int32),
        pltpu.VMEM((N + _L,), f32),
        pltpu.VMEM((N + _L,), f32),
        pltpu.VMEM((_QPW, _L), f32),
        pltpu.VMEM((_QPW, _L), f32),
        pltpu.VMEM((_QPW, _L), f32),
        pltpu.VMEM((_QPW, _L), f32),
        pltpu.SemaphoreType.DMA,
        pltpu.SemaphoreType.DMA,
        pltpu.SemaphoreType.DMA,
        pltpu.SemaphoreType.DMA,
        pltpu.SemaphoreType.DMA,
        pltpu.VMEM((_QPW, _L), f32),
        pltpu.VMEM((_QPW, _L), f32),
        pltpu.VMEM((_QPW, _L), f32),
        pltpu.VMEM((_QPW, _L), f32),
        pltpu.VMEM((_QPW, _L), f32),
        pltpu.VMEM((_QPW, _L), f32),
    ]
    run = pl.kernel(_body, out_type=out_type, mesh=mesh,
                    scratch_types=scratch_types,
                    compiler_params=pltpu.CompilerParams(
                        needs_layout_passes=False,
                        use_tc_tiling_on_sc=False))
    nf, gx, bq, _ = run(pts, qs, feats)
    return (nf, gx, bq)


# speedup vs baseline: 1.0338x; 1.0082x over previous
import jax
import jax.numpy as jnp
from jax import lax
from jax.experimental import pallas as pl
from jax.experimental.pallas import tpu as pltpu
from jax.experimental.pallas import tpu_sc as plsc

_RADIUS = 0.2
_NSAMPLE = 16
_N = 8192
_NQ = 2048
_B = 2
_CR = 128
_L = 16
_NC, _NSUB = 2, 16
_NW = _NC * _NSUB
_QPW = (_B * _NQ) // _NW
_NVREG = _N // _L
_HALF = _CR // 2


def _body(p_h, q_h, f_h,
          nf_h, gx_h, bq_h, ig_h,
          pflat, pxv, pyv, pzv, qflat, idxg, idxg2, cand, bqs,
          planeA, planeB, stA, stA2, stB, stB2,
          semLA, semLB, semSA, semSB, semW,
          gxx, gxy, gxz, xfx, xfy, xfz):
    cid = lax.axis_index("c")
    sid = lax.axis_index("s")
    wid = sid * _NC + cid
    q0 = wid * _QPW
    b = q0 // _NQ
    qb = q0 - b * _NQ
    widp = (sid ^ 1) * _NC + cid
    qbp = widp * _QPW - b * _NQ

    pltpu.sync_copy(p_h.at[b], pflat)
    pltpu.sync_copy(q_h.at[b, pl.ds(qb * 3, _QPW * 3)], qflat)

    lane = jnp.arange(_L, dtype=jnp.int32)
    r2 = jnp.float32(_RADIUS * _RADIUS)

    def _splat(ref, i):
        return plsc.load_gather(ref, [jnp.full((_L,), i, dtype=jnp.int32)])

    def deint(v, carry):
        base = v * _L
        i3 = (lane + base) * 3
        pxv[pl.ds(base, _L)] = plsc.load_gather(pflat, [i3])
        pyv[pl.ds(base, _L)] = plsc.load_gather(pflat, [i3 + 1])
        pzv[pl.ds(base, _L)] = plsc.load_gather(pflat, [i3 + 2])
        return carry

    lax.fori_loop(0, _NVREG, deint, jnp.int32(0), unroll=4)

    shadow = jnp.full((_L,), 1000000.0, dtype=jnp.float32)
    pxv[pl.ds(_N, _L)] = shadow
    pyv[pl.ds(_N, _L)] = shadow
    pzv[pl.ds(_N, _L)] = shadow
    zeros = jnp.zeros((_L,), dtype=jnp.float32)
    planeA[pl.ds(_N, _L)] = zeros
    planeB[pl.ds(_N, _L)] = zeros

    _U = 8
    _NG = _NVREG // _U

    def ballq(i, carry):
        qx = _splat(qflat, 3 * i)
        qy = _splat(qflat, 3 * i + 1)
        qz = _splat(qflat, 3 * i + 2)

        def w_cond(st):
            g, cnt = st
            return jnp.logical_and(g < _NG, cnt < _NSAMPLE)

        def w_body(st):
            g, cnt = st
            base0 = g * (_U * _L)
            for u in range(_U):
                base = base0 + u * _L
                px = pxv[pl.ds(base, _L)]
                py = pyv[pl.ds(base, _L)]
                pz = pzv[pl.ds(base, _L)]
                dx = qx - px
                dy = qy - py
                dz = qz - pz
                d2 = dx * dx + dy * dy + dz * dz
                m = d2 < r2
                plsc.store_compressed(cand.at[pl.ds(cnt, _L)],
                                      lane + base, mask=m)
                pc = plsc.all_reduce_population_count(m)
                cnt = cnt + pc[0]
            return g + 1, cnt

        _, cnt = lax.while_loop(
            w_cond, w_body, (jnp.int32(0), jnp.int32(0)))

        row = cand[pl.ds(0, _L)]
        firstv = _splat(cand, jnp.int32(0))
        first = jnp.where(cnt > 0, firstv, jnp.int32(0))
        bq_row = jnp.where(lane < cnt, row, first)
        pad = jnp.logical_and(lane > 0, lane >= cnt)
        bqs[i] = bq_row
        idxg[i] = jnp.where(pad, jnp.int32(_N), bq_row)
        return carry

    lax.fori_loop(0, _QPW, ballq, jnp.int32(0))
    pltpu.async_copy(bqs, bq_h.at[b, pl.ds(qb, _QPW)], semW)
    pltpu.sync_copy(idxg, ig_h.at[b, pl.ds(qb, _QPW)])

    def xyzg(i, carry):
        idxv = idxg[i]
        qx = _splat(qflat, 3 * i)
        qy = _splat(qflat, 3 * i + 1)
        qz = _splat(qflat, 3 * i + 2)
        gx = plsc.load_gather(pxv, [idxv]) - qx
        gy = plsc.load_gather(pyv, [idxv]) - qy
        gz = plsc.load_gather(pzv, [idxv]) - qz
        gxx[i] = gx
        gxy[i] = gy
        gxz[i] = gz
        thr = jnp.float32(100000.0)
        zero = jnp.float32(0.0)
        rad = jnp.float32(_RADIUS)
        xfx[i] = jnp.where(gx > thr, zero, gx) / rad
        xfy[i] = jnp.where(gy > thr, zero, gy) / rad
        xfz[i] = jnp.where(gz > thr, zero, gz) / rad
        return carry

    lax.fori_loop(0, _QPW, xyzg, jnp.int32(0), unroll=2)
    qsl = pl.ds(qb, _QPW)
    pltpu.async_copy(gxx, gx_h.at[b, 0, qsl], semW)
    pltpu.async_copy(gxy, gx_h.at[b, 1, qsl], semW)
    pltpu.async_copy(gxz, gx_h.at[b, 2, qsl], semW)
    for r in range(4):
        pltpu.async_copy(xfx, nf_h.at[b, 0, r, qsl], semW)
        pltpu.async_copy(xfy, nf_h.at[b, 1, r, qsl], semW)
        pltpu.async_copy(xfz, nf_h.at[b, 2, r, qsl], semW)

    plsc.subcore_barrier()
    pltpu.sync_copy(ig_h.at[b, pl.ds(qbp, _QPW)], idxg2)

    def gq(plane, stage, iref):
        def qloop(i, c2):
            stage[i] = plsc.load_gather(plane, [iref[i]])
            return c2
        lax.fori_loop(0, _QPW, qloop, jnp.int32(0), unroll=4)

    pbase = (sid % 2) * _HALF

    def _nf_dst(p, qoff):
        return nf_h.at[b, 3 + p // 4, p % 4, pl.ds(qoff, _QPW)]

    def _pl_src(p):
        return f_h.at[b, p]

    npair = _HALF // 2
    pltpu.async_copy(_pl_src(pbase), planeA.at[pl.ds(0, _N)], semLA)

    def pair(j, carry):
        p = pbase + j * 2
        pltpu.make_async_copy(_pl_src(p), planeA.at[pl.ds(0, _N)],
                              semLA).wait()
        pltpu.async_copy(_pl_src(p + 1), planeB.at[pl.ds(0, _N)], semLB)

        @pl.when(j > 0)
        def _():
            pltpu.make_async_copy(stA, _nf_dst(p - 2, qb), semSA).wait()
            pltpu.make_async_copy(stA2, _nf_dst(p - 2, qbp), semSA).wait()

        gq(planeA, stA, idxg)
        gq(planeA, stA2, idxg2)
        pltpu.async_copy(stA, _nf_dst(p, qb), semSA)
        pltpu.async_copy(stA2, _nf_dst(p, qbp), semSA)

        pltpu.make_async_copy(_pl_src(p + 1), planeB.at[pl.ds(0, _N)],
                              semLB).wait()

        @pl.when(j + 1 < npair)
        def _():
            pltpu.async_copy(_pl_src(p + 2), planeA.at[pl.ds(0, _N)], semLA)

        @pl.when(j > 0)
        def _():
            pltpu.make_async_copy(stB, _nf_dst(p - 1, qb), semSB).wait()
            pltpu.make_async_copy(stB2, _nf_dst(p - 1, qbp), semSB).wait()

        gq(planeB, stB, idxg)
        gq(planeB, stB2, idxg2)
        pltpu.async_copy(stB, _nf_dst(p + 1, qb), semSB)
        pltpu.async_copy(stB2, _nf_dst(p + 1, qbp), semSB)
        return carry

    lax.fori_loop(0, npair, pair, jnp.int32(0))

    plast = pbase + _HALF - 2
    pltpu.make_async_copy(stA, _nf_dst(plast, qb), semSA).wait()
    pltpu.make_async_copy(stA2, _nf_dst(plast, qbp), semSA).wait()
    pltpu.make_async_copy(stB, _nf_dst(plast + 1, qb), semSB).wait()
    pltpu.make_async_copy(stB2, _nf_dst(plast + 1, qbp), semSB).wait()
    pltpu.make_async_copy(bqs, bq_h.at[b, pl.ds(qb, _QPW)], semW).wait()
    pltpu.make_async_copy(gxx, gx_h.at[b, 0, qsl], semW).wait()
    pltpu.make_async_copy(gxy, gx_h.at[b, 1, qsl], semW).wait()
    pltpu.make_async_copy(gxz, gx_h.at[b, 2, qsl], semW).wait()
    for r in range(4):
        pltpu.make_async_copy(xfx, nf_h.at[b, 0, r, qsl], semW).wait()
        pltpu.make_async_copy(xfy, nf_h.at[b, 1, r, qsl], semW).wait()
        pltpu.make_async_copy(xfz, nf_h.at[b, 2, r, qsl], semW).wait()


def kernel(xyz, new_xyz, features):
    B, N, _ = xyz.shape
    pts = xyz.reshape(B, N * 3)
    qs = new_xyz.reshape(B, _NQ * 3)
    feats = features.reshape(B, _CR, N)

    mesh = plsc.VectorSubcoreMesh(
        core_axis_name="c", subcore_axis_name="s",
        num_cores=_NC, num_subcores=_NSUB)

    f32 = jnp.float32
    out_type = (
        jax.ShapeDtypeStruct((B, 35, 4, _NQ, _NSAMPLE), f32),
        jax.ShapeDtypeStruct((B, 3, _NQ, _NSAMPLE), f32),
        jax.ShapeDtypeStruct((B, _NQ, _NSAMPLE), jnp.int32),
        jax.ShapeDtypeStruct((B, _NQ, _NSAMPLE), jnp.int32),
    )
    scratch_types = [
        pltpu.VMEM((N * 3,), f32),
        pltpu.VMEM((N + _L,), f32),
        pltpu.VMEM((N + _L,), f32),
        pltpu.VMEM((N + _L,), f32),
        pltpu.VMEM((_QPW * 3,), f32),
        pltpu.VMEM((_QPW, _L), jnp.int32),
        pltpu.VMEM((_QPW, _L), jnp.int32),
        pltpu.VMEM((160,), jnp.int32),
        pltpu.VMEM((_QPW, _L), jnp.int32),
        pltpu.VMEM((N + _L,), f32),
        pltpu.VMEM((N + _L,), f32),
        pltpu.VMEM((_QPW, _L), f32),
        pltpu.VMEM((_QPW, _L), f32),
        pltpu.VMEM((_QPW, _L), f32),
        pltpu.VMEM((_QPW, _L), f32),
        pltpu.SemaphoreType.DMA,
        pltpu.SemaphoreType.DMA,
        pltpu.SemaphoreType.DMA,
        pltpu.SemaphoreType.DMA,
        pltpu.SemaphoreType.DMA,
        pltpu.VMEM((_QPW, _L), f32),
        pltpu.VMEM((_QPW, _L), f32),
        pltpu.VMEM((_QPW, _L), f32),
        pltpu.VMEM((_QPW, _L), f32),
        pltpu.VMEM((_QPW, _L), f32),
        pltpu.VMEM((_QPW, _L), f32),
    ]
    run = pl.kernel(_body, out_type=out_type, mesh=mesh,
                    scratch_types=scratch_types,
                    compiler_params=pltpu.CompilerParams(
                        needs_layout_passes=False,
                        use_tc_tiling_on_sc=False))
    nf, gx, bq, _ = run(pts, qs, feats)
    return (nf, gx, bq)
